# baseline (speedup 1.0000x reference)
import jax
import jax.numpy as jnp
from jax.experimental import pallas as pl

_BATCH = 8
_N_AC = 20000
_N_SCORE = 2000
_N_PAD = 2048
_N_NMS = 1000
_N_NMS_PAD = 1024
_NMS_TH = 0.7


def _decode_rows(ac, dl):
    ha = ac[2:3] - ac[0:1]
    wa = ac[3:4] - ac[1:2]
    cya = ac[0:1] + 0.5 * ha
    cxa = ac[1:2] + 0.5 * wa
    cy = dl[0:1] * ha + cya
    cx = dl[1:2] * wa + cxa
    h = dl[2:3] * ha
    w = dl[3:4] * wa
    y1 = jnp.clip(cy - 0.5 * h, 0.0, 1.0)
    x1 = jnp.clip(cx - 0.5 * w, 0.0, 1.0)
    y2 = jnp.clip(cy + 0.5 * h, 0.0, 1.0)
    x2 = jnp.clip(cx + 0.5 * w, 0.0, 1.0)
    return y1, x1, y2, x2


def _decode_cols(ac, dl):
    ha = ac[:, 2:3] - ac[:, 0:1]
    wa = ac[:, 3:4] - ac[:, 1:2]
    cya = ac[:, 0:1] + 0.5 * ha
    cxa = ac[:, 1:2] + 0.5 * wa
    cy = dl[:, 0:1] * ha + cya
    cx = dl[:, 1:2] * wa + cxa
    h = dl[:, 2:3] * ha
    w = dl[:, 3:4] * wa
    y1 = jnp.clip(cy - 0.5 * h, 0.0, 1.0)
    x1 = jnp.clip(cx - 0.5 * w, 0.0, 1.0)
    y2 = jnp.clip(cy + 0.5 * h, 0.0, 1.0)
    x2 = jnp.clip(cx + 0.5 * w, 0.0, 1.0)
    return y1, x1, y2, x2


def _roi_kernel(acr_ref, dlr_ref, acc_ref, dlc_ref, out_ref):
    acr = acr_ref[0]
    dlr = dlr_ref[0]
    acc = acc_ref[0]
    dlc = dlc_ref[0]

    y1r, x1r, y2r, x2r = _decode_rows(acr, dlr)
    y1c, x1c, y2c, x2c = _decode_cols(acc, dlc)

    arear = (y2r - y1r) * (x2r - x1r)
    areac = (y2c - y1c) * (x2c - x1c)

    ih = jnp.maximum(jnp.minimum(y2c, y2r) - jnp.maximum(y1c, y1r), 0.0)
    iw = jnp.maximum(jnp.minimum(x2c, x2r) - jnp.maximum(x1c, x1r), 0.0)
    inter = ih * iw
    union = jnp.maximum(areac + arear - inter, 1e-8)
    iou = inter / union

    ii = jax.lax.broadcasted_iota(jnp.int32, (_N_PAD, _N_PAD), 0)
    jj = jax.lax.broadcasted_iota(jnp.int32, (_N_PAD, _N_PAD), 1)
    m = jnp.where((iou > _NMS_TH) & (ii < jj), 1.0, 0.0)

    jrow = jax.lax.broadcasted_iota(jnp.int32, (1, _N_PAD), 1)
    valid = jrow < _N_SCORE
    keep0 = jnp.where(valid, 1.0, 0.0)

    def cond(st):
        return st[1]

    def body(st):
        keep, _ = st
        sup = jax.lax.dot_general(
            keep, m, (((1,), (0,)), ((), ())),
            preferred_element_type=jnp.float32)
        knew = jnp.where(valid & (sup <= 0.0), 1.0, 0.0)
        return knew, jnp.any(knew != keep)

    keep, _ = jax.lax.while_loop(cond, body, (keep0, jnp.bool_(True)))

    lt = jnp.where(ii <= jj, 1.0, 0.0)
    pos = jax.lax.dot_general(
        keep, lt, (((1,), (0,)), ((), ())),
        preferred_element_type=jnp.float32) - 1.0

    ocol = jax.lax.broadcasted_iota(
        jnp.int32, (_N_NMS_PAD, 1), 0).astype(jnp.float32)
    onehot = jnp.where((pos == ocol) & (keep > 0.0), 1.0, 0.0)

    boxes_r = jnp.concatenate([y1r, x1r, y2r, x2r], axis=0)
    out = jax.lax.dot_general(
        boxes_r, onehot, (((1,), (1,)), ((), ())),
        preferred_element_type=jnp.float32,
        precision=jax.lax.Precision.HIGHEST)
    out_ref[0] = out


def kernel(rpn_prob, rpn_del, anchors):
    scores = jnp.squeeze(rpn_prob, axis=-1)
    iota = jax.lax.broadcasted_iota(jnp.int32, scores.shape, 1)
    _, idx_sorted = jax.lax.sort((-scores, iota), dimension=1, num_keys=1)
    top_idx = idx_sorted[:, :_N_SCORE]
    top_del = jnp.take_along_axis(rpn_del, top_idx[..., None], axis=1)
    top_ac = jnp.take(anchors, top_idx, axis=0)
    top_del = jnp.concatenate(
        [top_del[..., :2], jnp.exp(top_del[..., 2:])], axis=-1)

    pad = ((0, 0), (0, _N_PAD - _N_SCORE), (0, 0))
    ac_c = jnp.pad(top_ac, pad)
    dl_c = jnp.pad(top_del, pad)
    ac_r = jnp.transpose(ac_c, (0, 2, 1))
    dl_r = jnp.transpose(dl_c, (0, 2, 1))

    out = pl.pallas_call(
        _roi_kernel,
        grid=(_BATCH,),
        in_specs=[
            pl.BlockSpec((1, 4, _N_PAD), lambda b: (b, 0, 0)),
            pl.BlockSpec((1, 4, _N_PAD), lambda b: (b, 0, 0)),
            pl.BlockSpec((1, _N_PAD, 4), lambda b: (b, 0, 0)),
            pl.BlockSpec((1, _N_PAD, 4), lambda b: (b, 0, 0)),
        ],
        out_specs=pl.BlockSpec((1, 4, _N_NMS_PAD), lambda b: (b, 0, 0)),
        out_shape=jax.ShapeDtypeStruct((_BATCH, 4, _N_NMS_PAD), jnp.float32),
    )(ac_r, dl_r, ac_c, dl_c)

    return jnp.transpose(out, (0, 2, 1))[:, :_N_NMS, :]


# speedup vs baseline: 1.0548x; 1.0548x over previous
import jax
import jax.numpy as jnp
from jax.experimental import pallas as pl
from jax.experimental.pallas import tpu as pltpu

_BATCH = 8
_N_AC = 20000
_N_SCORE = 2000
_N_PAD = 2048
_N_NMS = 1000
_N_NMS_PAD = 1024
_NMS_TH = 0.7


def _decode_rows(ac, dl):
    ha = ac[2:3] - ac[0:1]
    wa = ac[3:4] - ac[1:2]
    cya = ac[0:1] + 0.5 * ha
    cxa = ac[1:2] + 0.5 * wa
    cy = dl[0:1] * ha + cya
    cx = dl[1:2] * wa + cxa
    h = dl[2:3] * ha
    w = dl[3:4] * wa
    y1 = jnp.clip(cy - 0.5 * h, 0.0, 1.0)
    x1 = jnp.clip(cx - 0.5 * w, 0.0, 1.0)
    y2 = jnp.clip(cy + 0.5 * h, 0.0, 1.0)
    x2 = jnp.clip(cx + 0.5 * w, 0.0, 1.0)
    return y1, x1, y2, x2


def _decode_cols(ac, dl):
    ha = ac[:, 2:3] - ac[:, 0:1]
    wa = ac[:, 3:4] - ac[:, 1:2]
    cya = ac[:, 0:1] + 0.5 * ha
    cxa = ac[:, 1:2] + 0.5 * wa
    cy = dl[:, 0:1] * ha + cya
    cx = dl[:, 1:2] * wa + cxa
    h = dl[:, 2:3] * ha
    w = dl[:, 3:4] * wa
    y1 = jnp.clip(cy - 0.5 * h, 0.0, 1.0)
    x1 = jnp.clip(cx - 0.5 * w, 0.0, 1.0)
    y2 = jnp.clip(cy + 0.5 * h, 0.0, 1.0)
    x2 = jnp.clip(cx + 0.5 * w, 0.0, 1.0)
    return y1, x1, y2, x2


_BLK = 256
_NBLK = _N_PAD // _BLK


def _roi_kernel(acr_ref, dlr_ref, acc_ref, dlc_ref, out_ref, m_ref):
    acr = acr_ref[0]
    dlr = dlr_ref[0]
    acc = acc_ref[0]
    dlc = dlc_ref[0]

    y1r, x1r, y2r, x2r = _decode_rows(acr, dlr)
    y1c, x1c, y2c, x2c = _decode_cols(acc, dlc)

    arear = (y2r - y1r) * (x2r - x1r)
    areac = (y2c - y1c) * (x2c - x1c)

    zeros_blk = jnp.zeros((_BLK, _BLK), jnp.float32)
    loc_i = jax.lax.broadcasted_iota(jnp.int32, (_BLK, _BLK), 0)
    loc_j = jax.lax.broadcasted_iota(jnp.int32, (_BLK, _BLK), 1)
    for r in range(_NBLK):
        r0, r1 = r * _BLK, (r + 1) * _BLK
        for c in range(_NBLK):
            c0, c1 = c * _BLK, (c + 1) * _BLK
            if c < r:
                m_ref[r0:r1, c0:c1] = zeros_blk
                continue
            ihb = jnp.maximum(
                jnp.minimum(y2c[r0:r1], y2r[:, c0:c1])
                - jnp.maximum(y1c[r0:r1], y1r[:, c0:c1]), 0.0)
            iwb = jnp.maximum(
                jnp.minimum(x2c[r0:r1], x2r[:, c0:c1])
                - jnp.maximum(x1c[r0:r1], x1r[:, c0:c1]), 0.0)
            inter = ihb * iwb
            union = jnp.maximum(
                areac[r0:r1] + arear[:, c0:c1] - inter, 1e-8)
            sup = (inter / union) > _NMS_TH
            if c == r:
                sup = sup & (loc_i < loc_j)
            m_ref[r0:r1, c0:c1] = jnp.where(sup, 1.0, 0.0)
    m = m_ref[...]

    jrow = jax.lax.broadcasted_iota(jnp.int32, (1, _N_PAD), 1)
    valid = jrow < _N_SCORE
    keep0 = jnp.where(valid, 1.0, 0.0)

    def cond(st):
        return st[1]

    def body(st):
        keep, _ = st
        sup = jax.lax.dot_general(
            keep, m, (((1,), (0,)), ((), ())),
            preferred_element_type=jnp.float32)
        knew = jnp.where(valid & (sup <= 0.0), 1.0, 0.0)
        return knew, jnp.any(knew != keep)

    keep, _ = jax.lax.while_loop(cond, body, (keep0, jnp.bool_(True)))

    ii = jax.lax.broadcasted_iota(jnp.int32, (_N_PAD, _N_PAD), 0)
    jj = jax.lax.broadcasted_iota(jnp.int32, (_N_PAD, _N_PAD), 1)
    lt = jnp.where(ii <= jj, 1.0, 0.0)
    pos = jax.lax.dot_general(
        keep, lt, (((1,), (0,)), ((), ())),
        preferred_element_type=jnp.float32) - 1.0

    ocol = jax.lax.broadcasted_iota(
        jnp.int32, (_N_NMS_PAD, 1), 0).astype(jnp.float32)
    onehot = jnp.where((pos == ocol) & (keep > 0.0), 1.0, 0.0)

    boxes_r = jnp.concatenate([y1r, x1r, y2r, x2r], axis=0)
    out = jax.lax.dot_general(
        boxes_r, onehot, (((1,), (1,)), ((), ())),
        preferred_element_type=jnp.float32,
        precision=jax.lax.Precision.HIGHEST)
    out_ref[0] = out


def kernel(rpn_prob, rpn_del, anchors):
    scores = jnp.squeeze(rpn_prob, axis=-1)
    _, top_idx = jax.lax.top_k(scores, _N_SCORE)
    top_del = jnp.take_along_axis(rpn_del, top_idx[..., None], axis=1)
    top_ac = jnp.take(anchors, top_idx, axis=0)
    top_del = jnp.concatenate(
        [top_del[..., :2], jnp.exp(top_del[..., 2:])], axis=-1)

    pad = ((0, 0), (0, _N_PAD - _N_SCORE), (0, 0))
    ac_c = jnp.pad(top_ac, pad)
    dl_c = jnp.pad(top_del, pad)
    ac_r = jnp.transpose(ac_c, (0, 2, 1))
    dl_r = jnp.transpose(dl_c, (0, 2, 1))

    out = pl.pallas_call(
        _roi_kernel,
        grid=(_BATCH,),
        in_specs=[
            pl.BlockSpec((1, 4, _N_PAD), lambda b: (b, 0, 0)),
            pl.BlockSpec((1, 4, _N_PAD), lambda b: (b, 0, 0)),
            pl.BlockSpec((1, _N_PAD, 4), lambda b: (b, 0, 0)),
            pl.BlockSpec((1, _N_PAD, 4), lambda b: (b, 0, 0)),
        ],
        out_specs=pl.BlockSpec((1, 4, _N_NMS_PAD), lambda b: (b, 0, 0)),
        out_shape=jax.ShapeDtypeStruct((_BATCH, 4, _N_NMS_PAD), jnp.float32),
        scratch_shapes=[pltpu.VMEM((_N_PAD, _N_PAD), jnp.float32)],
    )(ac_r, dl_r, ac_c, dl_c)

    return jnp.transpose(out, (0, 2, 1))[:, :_N_NMS, :]
